# Initial kernel scaffold; baseline (speedup 1.0000x reference)
import jax
import jax.numpy as jnp
from jax.experimental import pallas as pl


def kernel(x, lut):
    raise NotImplementedError("write your pallas kernel here")


# speedup vs baseline: 4.7695x; 4.7695x over previous
import functools
import math

import jax
import jax.numpy as jnp
from jax import lax
from jax.experimental import pallas as pl
from jax.experimental.pallas import tpu as pltpu
from jax.experimental.pallas import tpu_sc as plsc

D_MODEL = 128
SCALE = math.sqrt(float(D_MODEL))

NC = 2
NS = 16
NW = NC * NS

CHUNK = 128


def _prescale(lut):
    v, d = lut.shape
    block = 2000
    assert v % block == 0

    def body(l_ref, o_ref):
        o_ref[...] = l_ref[...] * SCALE

    return pl.pallas_call(
        body,
        grid=(v // block,),
        in_specs=[pl.BlockSpec((block, d), lambda i: (i, 0))],
        out_specs=pl.BlockSpec((block, d), lambda i: (i, 0)),
        out_shape=jax.ShapeDtypeStruct((v, d), lut.dtype),
    )(lut)


def _make_gather(n_idx):
    assert n_idx % (NW * CHUNK) == 0
    steps = n_idx // (NW * CHUNK)
    mesh = plsc.VectorSubcoreMesh(
        core_axis_name="c", subcore_axis_name="s",
        num_cores=NC, num_subcores=NS)

    @functools.partial(
        pl.kernel,
        out_type=jax.ShapeDtypeStruct((n_idx, D_MODEL), jnp.float32),
        mesh=mesh,
        scratch_types=[
            pltpu.VMEM((CHUNK,), jnp.int32),
            pltpu.VMEM((CHUNK, D_MODEL), jnp.float32),
            pltpu.SemaphoreType.DMA,
        ],
    )
    def gather(idx_hbm, table_hbm, out_hbm, idx_v, rows_v, gsem):
        wid = lax.axis_index("s") * NC + lax.axis_index("c")

        def step(j, carry):
            blk = wid * steps + j
            pltpu.sync_copy(idx_hbm.at[blk], idx_v)
            pltpu.async_copy(table_hbm.at[idx_v], rows_v, gsem).wait()
            pltpu.sync_copy(rows_v, out_hbm.at[pl.ds(blk * CHUNK, CHUNK)])
            return carry

        lax.fori_loop(0, steps, step, 0)

    return gather


def kernel(x, lut):
    b0, b1 = x.shape
    n_idx = b0 * b1
    idx = x.reshape(n_idx // CHUNK, CHUNK).astype(jnp.int32)
    table = _prescale(lut)
    out = _make_gather(n_idx)(idx, table)
    return out.reshape(b0, b1, D_MODEL)


# speedup vs baseline: 7.9140x; 1.6593x over previous
import functools
import math

import jax
import jax.numpy as jnp
from jax import lax
from jax.experimental import pallas as pl
from jax.experimental.pallas import tpu as pltpu
from jax.experimental.pallas import tpu_sc as plsc

D_MODEL = 128
SCALE = math.sqrt(float(D_MODEL))

NC = 2
NS = 16
NW = NC * NS

CHUNK = 128
NBUF = 4


def _prescale(lut):
    v, d = lut.shape
    block = 2000
    assert v % block == 0

    def body(l_ref, o_ref):
        o_ref[...] = l_ref[...] * SCALE

    return pl.pallas_call(
        body,
        grid=(v // block,),
        in_specs=[pl.BlockSpec((block, d), lambda i: (i, 0))],
        out_specs=pl.BlockSpec((block, d), lambda i: (i, 0)),
        out_shape=jax.ShapeDtypeStruct((v, d), lut.dtype),
    )(lut)


def _make_gather(n_idx):
    assert n_idx % (NW * CHUNK * NBUF) == 0
    steps = n_idx // (NW * CHUNK)
    ngroups = steps // NBUF
    mesh = plsc.VectorSubcoreMesh(
        core_axis_name="c", subcore_axis_name="s",
        num_cores=NC, num_subcores=NS)

    @functools.partial(
        pl.kernel,
        out_type=jax.ShapeDtypeStruct((n_idx, D_MODEL), jnp.float32),
        mesh=mesh,
        scratch_types=(
            [pltpu.VMEM((steps, CHUNK), jnp.int32)]
            + [pltpu.VMEM((CHUNK, D_MODEL), jnp.float32)] * NBUF
            + [pltpu.SemaphoreType.DMA] * (2 * NBUF + 1)
        ),
    )
    def gather(idx_hbm, table_hbm, out_hbm, idx_v, *bufs_and_sems):
        rows = bufs_and_sems[:NBUF]
        gsem = bufs_and_sems[NBUF:2 * NBUF]
        ssem = bufs_and_sems[2 * NBUF:3 * NBUF]
        isem = bufs_and_sems[3 * NBUF]
        wid = lax.axis_index("s") * NC + lax.axis_index("c")
        base = wid * steps

        pltpu.async_copy(idx_hbm.at[pl.ds(base, steps)], idx_v, isem).wait()

        def fire_gather(j, b):
            pltpu.async_copy(table_hbm.at[idx_v.at[j]], rows[b], gsem[b])

        def fire_store(j, b):
            pltpu.async_copy(
                rows[b], out_hbm.at[pl.ds((base + j) * CHUNK, CHUNK)], ssem[b])

        def wait_gather(j, b):
            pltpu.make_async_copy(
                table_hbm.at[idx_v.at[j]], rows[b], gsem[b]).wait()

        def wait_store(j, b):
            pltpu.make_async_copy(
                rows[b], out_hbm.at[pl.ds((base + j) * CHUNK, CHUNK)],
                ssem[b]).wait()

        for b in range(NBUF):
            fire_gather(b, b)

        def group(g, carry):
            j0 = g * NBUF
            for b in range(NBUF):
                wait_gather(j0 + b, b)
                fire_store(j0 + b, b)

            @pl.when(g < ngroups - 1)
            def _():
                for b in range(NBUF):
                    wait_store(j0 + b, b)
                    fire_gather(j0 + NBUF + b, b)

            return carry

        lax.fori_loop(0, ngroups, group, 0)
        j_last = (ngroups - 1) * NBUF
        for b in range(NBUF):
            wait_store(j_last + b, b)

    return gather


def kernel(x, lut):
    b0, b1 = x.shape
    n_idx = b0 * b1
    idx = x.reshape(n_idx // CHUNK, CHUNK).astype(jnp.int32)
    table = _prescale(lut)
    out = _make_gather(n_idx)(idx, table)
    return out.reshape(b0, b1, D_MODEL)


# speedup vs baseline: 9.0912x; 1.1487x over previous
import functools
import math

import jax
import jax.numpy as jnp
from jax import lax
from jax.experimental import pallas as pl
from jax.experimental.pallas import tpu as pltpu
from jax.experimental.pallas import tpu_sc as plsc

D_MODEL = 128
SCALE = math.sqrt(float(D_MODEL))

NC = 2
NS = 16
NW = NC * NS

CHUNK = 128
NBUF = 4
LANES = 16


def _make_gather(n_idx):
    assert n_idx % (NW * CHUNK * NBUF) == 0
    steps = n_idx // (NW * CHUNK)
    ngroups = steps // NBUF
    vecs_per_row = D_MODEL // LANES
    mesh = plsc.VectorSubcoreMesh(
        core_axis_name="c", subcore_axis_name="s",
        num_cores=NC, num_subcores=NS)

    @functools.partial(
        pl.kernel,
        out_type=jax.ShapeDtypeStruct((n_idx, D_MODEL), jnp.float32),
        mesh=mesh,
        scratch_types=(
            [pltpu.VMEM((steps, CHUNK), jnp.int32)]
            + [pltpu.VMEM((CHUNK, D_MODEL), jnp.float32)] * NBUF
            + [pltpu.SemaphoreType.DMA] * (2 * NBUF + 1)
        ),
    )
    def gather(idx_hbm, table_hbm, out_hbm, idx_v, *bufs_and_sems):
        rows = bufs_and_sems[:NBUF]
        gsem = bufs_and_sems[NBUF:2 * NBUF]
        ssem = bufs_and_sems[2 * NBUF:3 * NBUF]
        isem = bufs_and_sems[3 * NBUF]
        wid = lax.axis_index("s") * NC + lax.axis_index("c")
        base = wid * steps

        pltpu.async_copy(idx_hbm.at[pl.ds(base, steps)], idx_v, isem).wait()

        def fire_gather(j, b):
            pltpu.async_copy(table_hbm.at[idx_v.at[j]], rows[b], gsem[b])

        def fire_store(j, b):
            pltpu.async_copy(
                rows[b], out_hbm.at[pl.ds((base + j) * CHUNK, CHUNK)], ssem[b])

        def wait_gather(j, b):
            pltpu.make_async_copy(
                table_hbm.at[idx_v.at[j]], rows[b], gsem[b]).wait()

        def wait_store(j, b):
            pltpu.make_async_copy(
                rows[b], out_hbm.at[pl.ds((base + j) * CHUNK, CHUNK)],
                ssem[b]).wait()

        def scale_buf(b):
            buf = rows[b]

            @plsc.parallel_loop(0, CHUNK, 1, unroll=4)
            def _(r):
                for u in range(vecs_per_row):
                    sl = pl.ds(u * LANES, LANES)
                    buf[r, sl] = buf[r, sl] * SCALE

        for b in range(NBUF):
            fire_gather(b, b)

        def group(g, carry):
            j0 = g * NBUF
            for b in range(NBUF):
                wait_gather(j0 + b, b)
                scale_buf(b)
                fire_store(j0 + b, b)

            @pl.when(g < ngroups - 1)
            def _():
                for b in range(NBUF):
                    wait_store(j0 + b, b)
                    fire_gather(j0 + NBUF + b, b)

            return carry

        lax.fori_loop(0, ngroups, group, 0)
        j_last = (ngroups - 1) * NBUF
        for b in range(NBUF):
            wait_store(j_last + b, b)

    return gather


def kernel(x, lut):
    b0, b1 = x.shape
    n_idx = b0 * b1
    idx = x.reshape(n_idx // CHUNK, CHUNK).astype(jnp.int32)
    out = _make_gather(n_idx)(idx, lut)
    return out.reshape(b0, b1, D_MODEL)
